# baseline (speedup 1.0000x reference)
import functools

import jax
import jax.numpy as jnp
from jax import lax
from jax.experimental import pallas as pl
from jax.experimental.pallas import tpu as pltpu
from jax.experimental.pallas import tpu_sc as plsc

N = 10000
E = 320000
ET = E + N
NW = 32
SPAN = 10320
EPAD = NW * SPAN
NB = 32
BSZ = 320
NPAD = NB * BSZ
CH = 256
ECAP = EPAD + NB * CH
EALLOC = ECAP + CH
DUMMY = ECAP + 128
NVEC = SPAN // 16
SROWS = (SPAN + 127) // 128

_mesh = plsc.VectorSubcoreMesh(core_axis_name="c", subcore_axis_name="s")
_sc_params = pltpu.CompilerParams(needs_layout_passes=False,
                                  use_tc_tiling_on_sc=False)


def _wid():
    return lax.axis_index("s") * 2 + lax.axis_index("c")


def _iota():
    return lax.iota(jnp.int32, 16)


def _take16(x, idx):
    return lax.gather(
        x, idx[:, None],
        lax.GatherDimensionNumbers(offset_dims=(), collapsed_slice_dims=(0,),
                                   start_index_map=(0,)),
        (1,), mode=lax.GatherScatterMode.PROMISE_IN_BOUNDS)


def _group_info(sd):
    k = _iota()
    prev = _take16(sd, jnp.maximum(k - 1, 0))
    is_start = (k == 0) | (sd != prev)
    startpos = plsc.cummax(jnp.where(is_start, k, 0))
    rank = k - startpos
    nxt = _take16(sd, jnp.minimum(k + 1, 15))
    is_end = (k == 15) | (sd != nxt)
    return rank, is_end


def _hist_body(dst_hbm, counts_hbm, dbuf, cnt):
    w = _wid()
    z16 = jnp.zeros((16,), jnp.int32)

    def zloop(i, _):
        cnt[pl.ds(i * 16, 16)] = z16
        return 0
    lax.fori_loop(0, NPAD // 16, zloop, 0)
    pltpu.sync_copy(dst_hbm.at[pl.ds(w * SPAN, SPAN)], dbuf)

    def body(i, _):
        d = dbuf[pl.ds(i * 16, 16)]
        sd, _sl = plsc.sort_key_val(d, _iota())
        rank, is_end = _group_info(sd)
        plsc.addupdate_scatter(cnt, [sd], rank + 1, mask=is_end)
        return 0
    lax.fori_loop(0, NVEC, body, 0)
    pltpu.sync_copy(cnt, counts_hbm.at[w])


_hist = pl.kernel(
    _hist_body,
    out_type=jax.ShapeDtypeStruct((NW, NPAD), jnp.int32),
    mesh=_mesh,
    compiler_params=_sc_params,
    scratch_types=[pltpu.VMEM((SPAN,), jnp.int32),
                   pltpu.VMEM((NPAD,), jnp.int32)],
)


def _place_body(src_hbm, dst_hbm, counts_hbm,
                ssrc_hbm, sdst_hbm, seid_hbm, nstart_hbm, ntot_hbm,
                sbuf, dbuf, tmp, tot, below, nstart_v, bstart_v,
                posb, soutb, doutb, eoutb, sem):
    w = _wid()
    z16 = jnp.zeros((16,), jnp.int32)
    nv = NPAD // 16

    def zloop(i, _):
        tot[pl.ds(i * 16, 16)] = z16
        below[pl.ds(i * 16, 16)] = z16
        return 0
    lax.fori_loop(0, nv, zloop, 0)

    def agg(t, _):
        pltpu.sync_copy(counts_hbm.at[t], tmp)

        def add(i, _):
            v = tmp[pl.ds(i * 16, 16)]
            tot[pl.ds(i * 16, 16)] += v
            return 0
        lax.fori_loop(0, nv, add, 0)

        @pl.when(t < w)
        def _():
            def addb(i, _):
                below[pl.ds(i * 16, 16)] += tmp[pl.ds(i * 16, 16)]
                return 0
            lax.fori_loop(0, nv, addb, 0)
        return 0
    lax.fori_loop(0, NW, agg, 0)

    def btot(b, run):
        def acc(i, a):
            return a + tot[pl.ds(b * BSZ + i * 16, 16)]
        a16 = lax.fori_loop(0, BSZ // 16, acc, z16)
        bt = jnp.sum(a16)
        bstart_v[b] = run
        return run + ((bt + CH - 1) & -CH)
    lax.fori_loop(0, NB, btot, jnp.int32(0))

    def nloop(b, _):
        bs = bstart_v[b]

        def inner(i, run):
            v = tot[pl.ds(b * BSZ + i * 16, 16)]
            c = plsc.cumsum(v)
            nstart_v[pl.ds(b * BSZ + i * 16, 16)] = c - v + run
            return run + jnp.sum(v)
        lax.fori_loop(0, BSZ // 16, inner, bs)
        return 0
    lax.fori_loop(0, NB, nloop, 0)

    def curs(i, _):
        below[pl.ds(i * 16, 16)] += nstart_v[pl.ds(i * 16, 16)]
        return 0
    lax.fori_loop(0, nv, curs, 0)

    @pl.when(w == 0)
    def _():
        pltpu.sync_copy(nstart_v, nstart_hbm)
        pltpu.sync_copy(tot, ntot_hbm)

    pltpu.sync_copy(src_hbm.at[pl.ds(w * SPAN, SPAN)], sbuf)
    pltpu.sync_copy(dst_hbm.at[pl.ds(w * SPAN, SPAN)], dbuf)
    dm16 = jnp.full((16,), DUMMY, jnp.int32)
    for c in range(8):
        posb[SROWS - 1, pl.ds(c * 16, 16)] = dm16

    def place(i, _):
        d = dbuf[pl.ds(i * 16, 16)]
        s = sbuf[pl.ds(i * 16, 16)]
        sd, sl = plsc.sort_key_val(d, _iota())
        sp = _take16(s, sl)
        ep = w * SPAN + i * 16 + sl
        rank, is_end = _group_info(sd)
        pos = plsc.load_gather(below, [sd]) + rank
        plsc.store_scatter(below, [sd], pos + 1, mask=is_end)
        r = i // 8
        cofs = (i % 8) * 16
        posb[r, pl.ds(cofs, 16)] = pos
        soutb[r, pl.ds(cofs, 16)] = sp
        doutb[r, pl.ds(cofs, 16)] = sd
        eoutb[r, pl.ds(cofs, 16)] = ep
        return 0
    lax.fori_loop(0, NVEC, place, 0)

    def scat(j, _):
        a = pltpu.async_copy(soutb.at[j], ssrc_hbm.at[posb.at[j]], sem)
        b = pltpu.async_copy(doutb.at[j], sdst_hbm.at[posb.at[j]], sem)
        c = pltpu.async_copy(eoutb.at[j], seid_hbm.at[posb.at[j]], sem)
        a.wait()
        b.wait()
        c.wait()
        return 0
    lax.fori_loop(0, SROWS, scat, 0)


_place = pl.kernel(
    _place_body,
    out_type=(jax.ShapeDtypeStruct((EALLOC,), jnp.int32),
              jax.ShapeDtypeStruct((EALLOC,), jnp.int32),
              jax.ShapeDtypeStruct((EALLOC,), jnp.int32),
              jax.ShapeDtypeStruct((NPAD,), jnp.int32),
              jax.ShapeDtypeStruct((NPAD,), jnp.int32)),
    mesh=_mesh,
    compiler_params=_sc_params,
    scratch_types=[pltpu.VMEM((SPAN,), jnp.int32),
                   pltpu.VMEM((SPAN,), jnp.int32),
                   pltpu.VMEM((NPAD,), jnp.int32),
                   pltpu.VMEM((NPAD,), jnp.int32),
                   pltpu.VMEM((NPAD,), jnp.int32),
                   pltpu.VMEM((NPAD,), jnp.int32),
                   pltpu.SMEM((NB,), jnp.int32),
                   pltpu.VMEM((SROWS, 128), jnp.int32),
                   pltpu.VMEM((SROWS, 128), jnp.int32),
                   pltpu.VMEM((SROWS, 128), jnp.int32),
                   pltpu.VMEM((SROWS, 128), jnp.int32),
                   pltpu.SemaphoreType.DMA],
)


def _msg1_body(ssrc, sdst, seid, nstart, ntot, aa, htab,
               msg, a1out, araw, anorm,
               nst_v, ntt_v, aa_loc, denom, src_c, dst_c, eid_c,
               asrc_r, araw_c, an_c, ai, gidx, hrows, acc, sem):
    w = _wid()
    nbase = pl.multiple_of(w * BSZ, BSZ)
    k16 = _iota()
    sel = k16 >> 3
    lane8 = k16 & 7
    z16 = jnp.zeros((16,), jnp.float32)
    pltpu.sync_copy(nstart.at[pl.ds(nbase, BSZ)], nst_v)
    pltpu.sync_copy(ntot.at[pl.ds(nbase, BSZ)], ntt_v)
    pltpu.sync_copy(aa.at[pl.ds(nbase, BSZ)], aa_loc)

    def cnt(i, a):
        return a + ntt_v[pl.ds(i * 16, 16)]
    necnt = jnp.sum(lax.fori_loop(0, BSZ // 16, cnt,
                                  jnp.zeros((16,), jnp.int32)))
    bstart = pl.multiple_of(nst_v[pl.ds(0, 16)][0], CH)
    nchunks = (necnt + CH - 1) // CH

    def zden(i, _):
        denom[pl.ds(i * 16, 16)] = z16
        return 0
    lax.fori_loop(0, (BSZ * 8) // 16, zden, 0)

    def load_chunk(j, with_eid):
        cofs = pl.multiple_of(bstart + j * CH, CH)
        pltpu.sync_copy(ssrc.at[pl.ds(cofs, CH)], src_c)
        pltpu.sync_copy(sdst.at[pl.ds(cofs, CH)], dst_c.at[pl.ds(0, CH)])
        if with_eid:
            pltpu.sync_copy(seid.at[pl.ds(cofs, CH)], eid_c)
        clen = jnp.minimum(CH, necnt - j * CH)

        def san(v, _):
            m = (v * 16 + k16) < clen
            sl = pl.ds(v * 16, 16)
            src_c[sl] = jnp.where(m, src_c[sl], 0)
            dst_c[sl] = jnp.where(m, dst_c[sl], nbase)
            if with_eid:
                eid_c[sl] = jnp.where(m, eid_c[sl], ET)
            return 0
        lax.fori_loop(0, CH // 16, san, 0)
        return cofs

    def s1(j, _):
        cofs = load_chunk(j, True)
        pltpu.async_copy(aa.at[src_c], asrc_r, sem).wait()

        def pair(i, _):
            for u in range(4):
                k = i * 4 + u
                ep = 2 * k + sel
                dl = plsc.load_gather(dst_c, [ep]) - nbase
                eidp = plsc.load_gather(eid_c, [ep])
                a_s = plsc.load_gather(asrc_r, [ep, lane8])
                a_d = plsc.load_gather(aa_loc, [dl, 8 + lane8])
                z = a_s + a_d
                z = jnp.where(z > 0, z, 0.2 * z)
                al = jnp.where(eidp < ET, jnp.exp(z), 0.0)
                araw_c[pl.ds(pl.multiple_of(k * 16, 16), 16)] = al
                addr = dl * 8 + lane8
                plsc.addupdate_scatter(denom, [addr], al, mask=(sel == 0))
                plsc.addupdate_scatter(denom, [addr], al, mask=(sel == 1))
            return 0
        lax.fori_loop(0, CH // 8, pair, 0)
        pltpu.sync_copy(araw_c, araw.at[pl.ds(cofs * 8, CH * 8)])
        return 0
    lax.fori_loop(0, nchunks, s1, 0)

    def s2(j, _):
        cofs = load_chunk(j, True)
        pltpu.sync_copy(araw.at[pl.ds(cofs * 8, CH * 8)], araw_c)

        def pair(i, _):
            for u in range(4):
                k = i * 4 + u
                ep = 2 * k + sel
                dl = plsc.load_gather(dst_c, [ep]) - nbase
                dv = plsc.load_gather(denom, [dl * 8 + lane8])
                ko = pl.multiple_of(k * 16, 16)
                al = araw_c[pl.ds(ko, 16)] / (dv + 1e-16)
                an_c[pl.ds(ko, 16)] = al
                eidp = plsc.load_gather(eid_c, [ep])
                ai[pl.ds(ko, 16)] = eidp * 8 + lane8
            return 0
        lax.fori_loop(0, CH // 8, pair, 0)
        pltpu.sync_copy(an_c.at[pl.ds(0, CH * 8)],
                        anorm.at[pl.ds(cofs * 8, CH * 8)])
        pltpu.async_copy(an_c.at[pl.ds(0, CH * 8)],
                         a1out.at[ai], sem).wait()
        return 0
    lax.fori_loop(0, nchunks, s2, 0)

    def sh(h, _):
        hfull = jnp.full((16,), h, jnp.int32)

        def zacc(i, _):
            for c in range(8):
                acc[i, pl.ds(c * 16, 16)] = z16
            return 0
        lax.fori_loop(0, BSZ, zacc, 0)

        def s3(j, _):
            cofs = load_chunk(j, False)

            def gi(v, _):
                sl = pl.ds(v * 16, 16)
                gidx[sl] = src_c[sl] * 8 + h
                return 0
            lax.fori_loop(0, CH // 16, gi, 0)
            cp = pltpu.async_copy(htab.at[gidx], hrows, sem)
            pltpu.sync_copy(anorm.at[pl.ds(cofs * 8, CH * 8)],
                            an_c.at[pl.ds(0, CH * 8)])
            cp.wait()

            def edge(i, _):
                dvec = dst_c[pl.ds(i * 8, 16)]
                for u in range(8):
                    e = i * 8 + u
                    dl = dvec[u] - nbase
                    ao = pl.multiple_of(e * 8, 8)
                    ab = _take16(an_c[pl.ds(ao, 16)], hfull)
                    for c in range(8):
                        sl = pl.ds(c * 16, 16)
                        plsc.addupdate(acc.at[dl, sl], ab * hrows[e, sl])
                return 0
            lax.fori_loop(0, CH // 8, edge, 0)
            return 0
        lax.fori_loop(0, nchunks, s3, 0)
        pltpu.sync_copy(acc, msg.at[pl.ds(nbase, BSZ), pl.ds(h * 128, 128)])
        return 0
    lax.fori_loop(0, 8, sh, 0)


_msg1 = pl.kernel(
    _msg1_body,
    out_type=(jax.ShapeDtypeStruct((NPAD, 1024), jnp.float32),
              jax.ShapeDtypeStruct((EPAD * 8,), jnp.float32),
              jax.ShapeDtypeStruct((ECAP * 8,), jnp.float32),
              jax.ShapeDtypeStruct((ECAP * 8,), jnp.float32)),
    mesh=_mesh,
    compiler_params=_sc_params,
    scratch_types=[pltpu.VMEM((BSZ,), jnp.int32),
                   pltpu.VMEM((BSZ,), jnp.int32),
                   pltpu.VMEM((BSZ, 16), jnp.float32),
                   pltpu.VMEM((BSZ * 8,), jnp.float32),
                   pltpu.VMEM((CH,), jnp.int32),
                   pltpu.VMEM((CH + 16,), jnp.int32),
                   pltpu.VMEM((CH,), jnp.int32),
                   pltpu.VMEM((CH, 16), jnp.float32),
                   pltpu.VMEM((CH * 8,), jnp.float32),
                   pltpu.VMEM((CH * 8 + 16,), jnp.float32),
                   pltpu.VMEM((CH * 8,), jnp.int32),
                   pltpu.VMEM((CH,), jnp.int32),
                   pltpu.VMEM((CH, 128), jnp.float32),
                   pltpu.VMEM((BSZ, 128), jnp.float32),
                   pltpu.SemaphoreType.DMA],
)


def _msg2_body(ssrc, sdst, seid, nstart, ntot, aa, htab,
               msg, a2out, araw,
               nst_v, ntt_v, aa_loc, denom, src_c, dst_c, eid_c,
               asrc_r, araw_c, an_c, ai, av, hrows, acc, sem):
    w = _wid()
    nbase = pl.multiple_of(w * BSZ, BSZ)
    k16 = _iota()
    z16 = jnp.zeros((16,), jnp.float32)
    zi16 = jnp.zeros((16,), jnp.int32)
    one16 = jnp.ones((16,), jnp.int32)
    pltpu.sync_copy(nstart.at[pl.ds(nbase, BSZ)], nst_v)
    pltpu.sync_copy(ntot.at[pl.ds(nbase, BSZ)], ntt_v)
    pltpu.sync_copy(aa.at[pl.ds(nbase, BSZ)], aa_loc)

    def cnt(i, a):
        return a + ntt_v[pl.ds(i * 16, 16)]
    necnt = jnp.sum(lax.fori_loop(0, BSZ // 16, cnt,
                                  jnp.zeros((16,), jnp.int32)))
    bstart = pl.multiple_of(nst_v[pl.ds(0, 16)][0], CH)
    nchunks = (necnt + CH - 1) // CH

    def zden(i, _):
        denom[pl.ds(i * 16, 16)] = z16
        return 0
    lax.fori_loop(0, BSZ // 16, zden, 0)

    def zacc(i, _):
        for c in range(4):
            acc[i, pl.ds(c * 16, 16)] = z16
        return 0
    lax.fori_loop(0, BSZ, zacc, 0)

    def load_chunk(j, with_eid):
        cofs = pl.multiple_of(bstart + j * CH, CH)
        pltpu.sync_copy(ssrc.at[pl.ds(cofs, CH)], src_c)
        pltpu.sync_copy(sdst.at[pl.ds(cofs, CH)], dst_c.at[pl.ds(0, CH)])
        if with_eid:
            pltpu.sync_copy(seid.at[pl.ds(cofs, CH)], eid_c)
        clen = jnp.minimum(CH, necnt - j * CH)

        def san(v, _):
            m = (v * 16 + k16) < clen
            sl = pl.ds(v * 16, 16)
            src_c[sl] = jnp.where(m, src_c[sl], 0)
            dst_c[sl] = jnp.where(m, dst_c[sl], nbase)
            if with_eid:
                eid_c[sl] = jnp.where(m, eid_c[sl], ET)
            return 0
        lax.fori_loop(0, CH // 16, san, 0)
        return cofs

    def s1(j, _):
        cofs = load_chunk(j, True)
        pltpu.async_copy(aa.at[src_c], asrc_r, sem).wait()

        def vec(v, _):
            sl = pl.ds(v * 16, 16)
            d16 = dst_c[sl]
            dl = d16 - nbase
            eid16 = eid_c[sl]
            a_s = plsc.load_gather(asrc_r, [v * 16 + k16, zi16])
            a_d = plsc.load_gather(aa_loc, [dl, one16])
            z = a_s + a_d
            z = jnp.where(z > 0, z, 0.2 * z)
            al = jnp.where(eid16 < ET, jnp.exp(z), 0.0)
            araw_c[sl] = al
            prev = _take16(d16, jnp.maximum(k16 - 1, 0))
            is_start = (k16 == 0) | (d16 != prev)
            csum = plsc.cumsum(al)
            spos = plsc.cummax(jnp.where(is_start, k16, 0))
            base_excl = jnp.where(spos > 0,
                                  _take16(csum, jnp.maximum(spos - 1, 0)),
                                  0.0)
            nxt = _take16(d16, jnp.minimum(k16 + 1, 15))
            is_end = (k16 == 15) | (d16 != nxt)
            plsc.addupdate_scatter(denom, [dl], csum - base_excl,
                                   mask=is_end)
            return 0
        lax.fori_loop(0, CH // 16, vec, 0)
        pltpu.sync_copy(araw_c, araw.at[pl.ds(cofs, CH)])
        return 0
    lax.fori_loop(0, nchunks, s1, 0)

    def s2(j, _):
        cofs = load_chunk(j, True)
        pltpu.sync_copy(araw.at[pl.ds(cofs, CH)], araw_c)
        pltpu.async_copy(htab.at[src_c], hrows, sem).wait()

        def vec(v, _):
            sl = pl.ds(v * 16, 16)
            dl = dst_c[sl] - nbase
            dv = plsc.load_gather(denom, [dl])
            aln = araw_c[sl] / (dv + 1e-16)
            an_c[sl] = aln
            r = v // 8
            c = (v % 8) * 16
            ai[r, pl.ds(c, 16)] = eid_c[sl]
            av[r, pl.ds(c, 16)] = aln
            return 0
        lax.fori_loop(0, CH // 16, vec, 0)
        d0 = pltpu.async_copy(av.at[0], a2out.at[ai.at[0]], sem)
        d1 = pltpu.async_copy(av.at[1], a2out.at[ai.at[1]], sem)

        def edge(i, _):
            io = pl.multiple_of(i * 8, 8)
            dvec = dst_c[pl.ds(io, 16)]
            avec = an_c[pl.ds(io, 16)]
            for u in range(8):
                e = i * 8 + u
                dl = dvec[u] - nbase
                ab = _take16(avec, jnp.full((16,), u, jnp.int32))
                for c in range(4):
                    sl = pl.ds(c * 16, 16)
                    acc[dl, sl] += ab * hrows[e, sl]
            return 0
        lax.fori_loop(0, CH // 8, edge, 0)
        d0.wait()
        d1.wait()
        return 0
    lax.fori_loop(0, nchunks, s2, 0)
    pltpu.sync_copy(acc, msg.at[pl.ds(nbase, BSZ)])


_msg2 = pl.kernel(
    _msg2_body,
    out_type=(jax.ShapeDtypeStruct((NPAD, 64), jnp.float32),
              jax.ShapeDtypeStruct((EPAD,), jnp.float32),
              jax.ShapeDtypeStruct((ECAP,), jnp.float32)),
    mesh=_mesh,
    compiler_params=_sc_params,
    scratch_types=[pltpu.VMEM((BSZ,), jnp.int32),
                   pltpu.VMEM((BSZ,), jnp.int32),
                   pltpu.VMEM((BSZ, 16), jnp.float32),
                   pltpu.VMEM((BSZ,), jnp.float32),
                   pltpu.VMEM((CH,), jnp.int32),
                   pltpu.VMEM((CH + 16,), jnp.int32),
                   pltpu.VMEM((CH,), jnp.int32),
                   pltpu.VMEM((CH, 16), jnp.float32),
                   pltpu.VMEM((CH,), jnp.float32),
                   pltpu.VMEM((CH + 16,), jnp.float32),
                   pltpu.VMEM((2, 128), jnp.int32),
                   pltpu.VMEM((2, 128), jnp.float32),
                   pltpu.VMEM((CH, 64), jnp.float32),
                   pltpu.VMEM((BSZ, 64), jnp.float32),
                   pltpu.SemaphoreType.DMA],
)


def _mm_kernel(x_ref, w_ref, o_ref):
    o_ref[...] = jnp.dot(x_ref[...], w_ref[...],
                         preferred_element_type=jnp.float32)


def _matmul(x, w, block_m=1000):
    M, K = x.shape
    _, Nc = w.shape
    return pl.pallas_call(
        _mm_kernel,
        grid=(M // block_m,),
        in_specs=[pl.BlockSpec((block_m, K), lambda i: (i, 0)),
                  pl.BlockSpec((K, Nc), lambda i: (0, 0))],
        out_specs=pl.BlockSpec((block_m, Nc), lambda i: (i, 0)),
        out_shape=jax.ShapeDtypeStruct((M, Nc), jnp.float32),
    )(x, w)


def _mm_elu_kernel(m_ref, b_ref, w_ref, o_ref):
    v = m_ref[...] + b_ref[...]
    v = jnp.where(v > 0, v, jnp.exp(v) - 1.0)
    o_ref[...] = jnp.dot(v, w_ref[...], preferred_element_type=jnp.float32)


def _mm_elu(msg, b, w, block_m=1000):
    K = msg.shape[1]
    Nc = w.shape[1]
    return pl.pallas_call(
        _mm_elu_kernel,
        grid=(N // block_m,),
        in_specs=[pl.BlockSpec((block_m, K), lambda i: (i, 0)),
                  pl.BlockSpec((1, K), lambda i: (0, 0)),
                  pl.BlockSpec((K, Nc), lambda i: (0, 0))],
        out_specs=pl.BlockSpec((block_m, Nc), lambda i: (i, 0)),
        out_shape=jax.ShapeDtypeStruct((N, Nc), jnp.float32),
    )(msg, b.reshape(1, K), w)


def _lsm_kernel(m_ref, b_ref, o_ref):
    z = m_ref[...] + b_ref[...]
    mx = jnp.max(z, axis=1, keepdims=True)
    s = jnp.log(jnp.sum(jnp.exp(z - mx), axis=1, keepdims=True))
    o_ref[...] = z - mx - s


def _logsoftmax(msg2, b2, block_m=1000):
    return pl.pallas_call(
        _lsm_kernel,
        grid=(N // block_m,),
        in_specs=[pl.BlockSpec((block_m, 64), lambda i: (i, 0)),
                  pl.BlockSpec((1, 64), lambda i: (0, 0))],
        out_specs=pl.BlockSpec((block_m, 64), lambda i: (i, 0)),
        out_shape=jax.ShapeDtypeStruct((N, 64), jnp.float32),
    )(msg2, b2.reshape(1, 64))


def kernel(x, edge_index, W1, att_src1, att_dst1, b1, W2, att_src2,
           att_dst2, b2):
    loop = jnp.arange(N, dtype=edge_index.dtype)
    src = jnp.concatenate(
        [edge_index[0], loop, jnp.zeros((EPAD - ET,), edge_index.dtype)])
    dst = jnp.concatenate(
        [edge_index[1], loop, jnp.full((EPAD - ET,), N - 1, edge_index.dtype)])

    counts = _hist(dst)
    ssrc_r, sdst_r, seid_r, nstart, ntot = _place(src, dst, counts)

    h1mat = _matmul(x, W1)
    eye8 = jnp.eye(8, dtype=jnp.float32)
    A_s = (att_src1.reshape(8, 128)[:, :, None] *
           eye8[:, None, :]).reshape(1024, 8)
    A_d = (att_dst1.reshape(8, 128)[:, :, None] *
           eye8[:, None, :]).reshape(1024, 8)
    aa1 = _matmul(h1mat, jnp.concatenate([A_s, A_d], axis=1))
    aa1p = jnp.pad(aa1, ((0, NPAD - N), (0, 0)))
    htab = h1mat.reshape(N * 8, 128)
    msg, a1flat, _araw, _anorm = _msg1(ssrc_r, sdst_r, seid_r, nstart,
                                       ntot, aa1p, htab)
    alpha1 = a1flat.reshape(EPAD, 8)[:ET]

    h2pre = _mm_elu(msg, b1, W2)
    A2 = jnp.concatenate([att_src2.reshape(64, 1), att_dst2.reshape(64, 1),
                          jnp.zeros((64, 14), jnp.float32)], axis=1)
    aa2p = jnp.pad(_matmul(h2pre, A2), ((0, NPAD - N), (0, 0)))
    msg2, a2flat, _araw2 = _msg2(ssrc_r, sdst_r, seid_r, nstart, ntot,
                                 aa2p, h2pre)
    logp = _logsoftmax(msg2, b2)
    alpha2 = a2flat[:ET].reshape(ET, 1)
    return (logp, alpha1, alpha2)


# speedup vs baseline: 2.0961x; 2.0961x over previous
import functools

import jax
import jax.numpy as jnp
from jax import lax
from jax.experimental import pallas as pl
from jax.experimental.pallas import tpu as pltpu
from jax.experimental.pallas import tpu_sc as plsc

N = 10000
E = 320000
ET = E + N
NW = 32
SPAN = 10320
EPAD = NW * SPAN
NB = 32
BSZ = 320
NPAD = NB * BSZ
CH = 256
ECAP = EPAD + NB * CH
EALLOC = ECAP + CH
DUMMY = ECAP + 128
NVEC = SPAN // 16
SROWS = (SPAN + 127) // 128

_mesh = plsc.VectorSubcoreMesh(core_axis_name="c", subcore_axis_name="s")
_sc_params = pltpu.CompilerParams(needs_layout_passes=False,
                                  use_tc_tiling_on_sc=False)


def _wid():
    return lax.axis_index("s") * 2 + lax.axis_index("c")


def _iota():
    return lax.iota(jnp.int32, 16)


def _take16(x, idx):
    return lax.gather(
        x, idx[:, None],
        lax.GatherDimensionNumbers(offset_dims=(), collapsed_slice_dims=(0,),
                                   start_index_map=(0,)),
        (1,), mode=lax.GatherScatterMode.PROMISE_IN_BOUNDS)


def _group_info(sd):
    k = _iota()
    prev = _take16(sd, jnp.maximum(k - 1, 0))
    is_start = (k == 0) | (sd != prev)
    startpos = plsc.cummax(jnp.where(is_start, k, 0))
    rank = k - startpos
    nxt = _take16(sd, jnp.minimum(k + 1, 15))
    is_end = (k == 15) | (sd != nxt)
    return rank, is_end


def _hist_body(dst_hbm, counts_hbm, dbuf, cnt):
    w = _wid()
    z16 = jnp.zeros((16,), jnp.int32)

    def zloop(i, _):
        cnt[pl.ds(i * 16, 16)] = z16
        return 0
    lax.fori_loop(0, NPAD // 16, zloop, 0)
    pltpu.sync_copy(dst_hbm.at[pl.ds(w * SPAN, SPAN)], dbuf)

    def body(i, _):
        d = dbuf[pl.ds(i * 16, 16)]
        sd, _sl = plsc.sort_key_val(d, _iota())
        rank, is_end = _group_info(sd)
        plsc.addupdate_scatter(cnt, [sd], rank + 1, mask=is_end)
        return 0
    lax.fori_loop(0, NVEC, body, 0)
    pltpu.sync_copy(cnt, counts_hbm.at[w])


_hist = pl.kernel(
    _hist_body,
    out_type=jax.ShapeDtypeStruct((NW, NPAD), jnp.int32),
    mesh=_mesh,
    compiler_params=_sc_params,
    scratch_types=[pltpu.VMEM((SPAN,), jnp.int32),
                   pltpu.VMEM((NPAD,), jnp.int32)],
)


def _place_body(src_hbm, dst_hbm, counts_hbm,
                ssrc_hbm, sdst_hbm, seid_hbm, nstart_hbm, ntot_hbm,
                sbuf, dbuf, tmp, tot, below, nstart_v, bstart_v,
                posb, soutb, doutb, eoutb, sem):
    w = _wid()
    z16 = jnp.zeros((16,), jnp.int32)
    nv = NPAD // 16

    def zloop(i, _):
        tot[pl.ds(i * 16, 16)] = z16
        below[pl.ds(i * 16, 16)] = z16
        return 0
    lax.fori_loop(0, nv, zloop, 0)

    def agg(t, _):
        pltpu.sync_copy(counts_hbm.at[t], tmp)

        def add(i, _):
            v = tmp[pl.ds(i * 16, 16)]
            tot[pl.ds(i * 16, 16)] += v
            return 0
        lax.fori_loop(0, nv, add, 0)

        @pl.when(t < w)
        def _():
            def addb(i, _):
                below[pl.ds(i * 16, 16)] += tmp[pl.ds(i * 16, 16)]
                return 0
            lax.fori_loop(0, nv, addb, 0)
        return 0
    lax.fori_loop(0, NW, agg, 0)

    def btot(b, run):
        def acc(i, a):
            return a + tot[pl.ds(b * BSZ + i * 16, 16)]
        a16 = lax.fori_loop(0, BSZ // 16, acc, z16)
        bt = jnp.sum(a16)
        bstart_v[b] = run
        return run + ((bt + CH - 1) & -CH)
    lax.fori_loop(0, NB, btot, jnp.int32(0))

    def nloop(b, _):
        bs = bstart_v[b]

        def inner(i, run):
            v = tot[pl.ds(b * BSZ + i * 16, 16)]
            c = plsc.cumsum(v)
            nstart_v[pl.ds(b * BSZ + i * 16, 16)] = c - v + run
            return run + jnp.sum(v)
        lax.fori_loop(0, BSZ // 16, inner, bs)
        return 0
    lax.fori_loop(0, NB, nloop, 0)

    def curs(i, _):
        below[pl.ds(i * 16, 16)] += nstart_v[pl.ds(i * 16, 16)]
        return 0
    lax.fori_loop(0, nv, curs, 0)

    @pl.when(w == 0)
    def _():
        pltpu.sync_copy(nstart_v, nstart_hbm)
        pltpu.sync_copy(tot, ntot_hbm)

    pltpu.sync_copy(src_hbm.at[pl.ds(w * SPAN, SPAN)], sbuf)
    pltpu.sync_copy(dst_hbm.at[pl.ds(w * SPAN, SPAN)], dbuf)
    dm16 = jnp.full((16,), DUMMY, jnp.int32)
    for c in range(8):
        posb[SROWS - 1, pl.ds(c * 16, 16)] = dm16

    def place(i, _):
        d = dbuf[pl.ds(i * 16, 16)]
        s = sbuf[pl.ds(i * 16, 16)]
        sd, sl = plsc.sort_key_val(d, _iota())
        sp = _take16(s, sl)
        ep = w * SPAN + i * 16 + sl
        rank, is_end = _group_info(sd)
        pos = plsc.load_gather(below, [sd]) + rank
        plsc.store_scatter(below, [sd], pos + 1, mask=is_end)
        r = i // 8
        cofs = (i % 8) * 16
        posb[r, pl.ds(cofs, 16)] = pos
        soutb[r, pl.ds(cofs, 16)] = sp
        doutb[r, pl.ds(cofs, 16)] = sd
        eoutb[r, pl.ds(cofs, 16)] = ep
        return 0
    lax.fori_loop(0, NVEC, place, 0)

    def scat(j, _):
        a = pltpu.async_copy(soutb.at[j], ssrc_hbm.at[posb.at[j]], sem)
        b = pltpu.async_copy(doutb.at[j], sdst_hbm.at[posb.at[j]], sem)
        c = pltpu.async_copy(eoutb.at[j], seid_hbm.at[posb.at[j]], sem)
        a.wait()
        b.wait()
        c.wait()
        return 0
    lax.fori_loop(0, SROWS, scat, 0)


_place = pl.kernel(
    _place_body,
    out_type=(jax.ShapeDtypeStruct((EALLOC,), jnp.int32),
              jax.ShapeDtypeStruct((EALLOC,), jnp.int32),
              jax.ShapeDtypeStruct((EALLOC,), jnp.int32),
              jax.ShapeDtypeStruct((NPAD,), jnp.int32),
              jax.ShapeDtypeStruct((NPAD,), jnp.int32)),
    mesh=_mesh,
    compiler_params=_sc_params,
    scratch_types=[pltpu.VMEM((SPAN,), jnp.int32),
                   pltpu.VMEM((SPAN,), jnp.int32),
                   pltpu.VMEM((NPAD,), jnp.int32),
                   pltpu.VMEM((NPAD,), jnp.int32),
                   pltpu.VMEM((NPAD,), jnp.int32),
                   pltpu.VMEM((NPAD,), jnp.int32),
                   pltpu.SMEM((NB,), jnp.int32),
                   pltpu.VMEM((SROWS, 128), jnp.int32),
                   pltpu.VMEM((SROWS, 128), jnp.int32),
                   pltpu.VMEM((SROWS, 128), jnp.int32),
                   pltpu.VMEM((SROWS, 128), jnp.int32),
                   pltpu.SemaphoreType.DMA],
)


def _msg1_body(ssrc, sdst, seid, nstart, ntot, aa, htab,
               msg, araw, dnout,
               nst_v, ntt_v, aa_loc, denom, src_c, dst_c, eid_c,
               asrc_r, araw_c, gidx, hrows, acc, sem):
    w = _wid()
    nbase = pl.multiple_of(w * BSZ, BSZ)
    k16 = _iota()
    sel = k16 >> 3
    lane8 = k16 & 7
    z16 = jnp.zeros((16,), jnp.float32)
    pltpu.sync_copy(nstart.at[pl.ds(nbase, BSZ)], nst_v)
    pltpu.sync_copy(ntot.at[pl.ds(nbase, BSZ)], ntt_v)
    pltpu.sync_copy(aa.at[pl.ds(nbase, BSZ)], aa_loc)

    def cnt(i, a):
        return a + ntt_v[pl.ds(i * 16, 16)]
    necnt = jnp.sum(lax.fori_loop(0, BSZ // 16, cnt,
                                  jnp.zeros((16,), jnp.int32)))
    bstart = pl.multiple_of(nst_v[pl.ds(0, 16)][0], CH)
    nchunks = (necnt + CH - 1) // CH

    def zden(i, _):
        denom[pl.ds(i * 16, 16)] = z16
        return 0
    lax.fori_loop(0, (BSZ * 8) // 16, zden, 0)

    def load_chunk(j, with_eid):
        cofs = pl.multiple_of(bstart + j * CH, CH)
        pltpu.sync_copy(ssrc.at[pl.ds(cofs, CH)], src_c)
        pltpu.sync_copy(sdst.at[pl.ds(cofs, CH)], dst_c.at[pl.ds(0, CH)])
        if with_eid:
            pltpu.sync_copy(seid.at[pl.ds(cofs, CH)], eid_c)
        clen = jnp.minimum(CH, necnt - j * CH)

        def san(v, _):
            m = (v * 16 + k16) < clen
            sl = pl.ds(v * 16, 16)
            src_c[sl] = jnp.where(m, src_c[sl], 0)
            dst_c[sl] = jnp.where(m, dst_c[sl], nbase)
            if with_eid:
                eid_c[sl] = jnp.where(m, eid_c[sl], ET)
            return 0
        lax.fori_loop(0, CH // 16, san, 0)
        return cofs

    def s1(j, _):
        cofs = load_chunk(j, True)
        pltpu.async_copy(aa.at[src_c], asrc_r, sem).wait()

        def pair(i, _):
            for u in range(4):
                k = i * 4 + u
                ep = 2 * k + sel
                dl = plsc.load_gather(dst_c, [ep]) - nbase
                eidp = plsc.load_gather(eid_c, [ep])
                a_s = plsc.load_gather(asrc_r, [ep, lane8])
                a_d = plsc.load_gather(aa_loc, [dl, 8 + lane8])
                z = a_s + a_d
                z = jnp.where(z > 0, z, 0.2 * z)
                al = jnp.where(eidp < ET, jnp.exp(z), 0.0)
                araw_c[pl.ds(pl.multiple_of(k * 16, 16), 16)] = al
                addr = dl * 8 + lane8
                plsc.addupdate_scatter(denom, [addr], al, mask=(sel == 0))
                plsc.addupdate_scatter(denom, [addr], al, mask=(sel == 1))
            return 0
        lax.fori_loop(0, CH // 8, pair, 0)
        pltpu.sync_copy(araw_c.at[pl.ds(0, CH * 8)],
                        araw.at[pl.ds(cofs * 8, CH * 8)])
        return 0
    lax.fori_loop(0, nchunks, s1, 0)
    pltpu.sync_copy(denom, dnout.at[pl.ds(nbase * 8, BSZ * 8)])


    def sh(h, _):
        hfull = jnp.full((16,), h, jnp.int32)

        def zacc(i, _):
            for c in range(8):
                acc[i, pl.ds(c * 16, 16)] = z16
            return 0
        lax.fori_loop(0, BSZ, zacc, 0)

        def s3(j, _):
            cofs = load_chunk(j, False)

            def gi(v, _):
                sl = pl.ds(v * 16, 16)
                gidx[sl] = src_c[sl] * 8 + h
                return 0
            lax.fori_loop(0, CH // 16, gi, 0)
            cp = pltpu.async_copy(htab.at[gidx], hrows, sem)
            pltpu.sync_copy(araw.at[pl.ds(cofs * 8, CH * 8)],
                            araw_c.at[pl.ds(0, CH * 8)])
            cp.wait()

            def edge(i, _):
                dvec = dst_c[pl.ds(i * 8, 16)]
                for u in range(8):
                    e = i * 8 + u
                    dl = dvec[u] - nbase
                    ao = pl.multiple_of(e * 8, 8)
                    ab = _take16(araw_c[pl.ds(ao, 16)], hfull)
                    for c in range(8):
                        sl = pl.ds(c * 16, 16)
                        plsc.addupdate(acc.at[dl, sl], ab * hrows[e, sl])
                return 0
            lax.fori_loop(0, CH // 8, edge, 0)
            return 0
        lax.fori_loop(0, nchunks, s3, 0)

        def nsc(n, _):
            dv = plsc.load_gather(denom,
                                  [jnp.full((16,), n, jnp.int32) * 8 + hfull])
            inv = 1.0 / (dv + 1e-16)
            for c in range(8):
                sl = pl.ds(c * 16, 16)
                acc[n, sl] *= inv
            return 0
        lax.fori_loop(0, BSZ, nsc, 0)
        pltpu.sync_copy(acc, msg.at[pl.ds(nbase, BSZ), pl.ds(h * 128, 128)])
        return 0
    lax.fori_loop(0, 8, sh, 0)


_msg1 = pl.kernel(
    _msg1_body,
    out_type=(jax.ShapeDtypeStruct((NPAD, 1024), jnp.float32),
              jax.ShapeDtypeStruct((ECAP * 8,), jnp.float32),
              jax.ShapeDtypeStruct((NPAD * 8,), jnp.float32)),
    mesh=_mesh,
    compiler_params=_sc_params,
    scratch_types=[pltpu.VMEM((BSZ,), jnp.int32),
                   pltpu.VMEM((BSZ,), jnp.int32),
                   pltpu.VMEM((BSZ, 16), jnp.float32),
                   pltpu.VMEM((BSZ * 8,), jnp.float32),
                   pltpu.VMEM((CH,), jnp.int32),
                   pltpu.VMEM((CH + 16,), jnp.int32),
                   pltpu.VMEM((CH,), jnp.int32),
                   pltpu.VMEM((CH, 16), jnp.float32),
                   pltpu.VMEM((CH * 8 + 16,), jnp.float32),
                   pltpu.VMEM((CH,), jnp.int32),
                   pltpu.VMEM((CH, 128), jnp.float32),
                   pltpu.VMEM((BSZ, 128), jnp.float32),
                   pltpu.SemaphoreType.DMA],
)


CPT = (SPAN + CH - 1) // CH


def _alpha1_body(srcp, dstp, aa, td, a1r, src_c, dst_c, asrc_r, adst_r,
                 an_c, sem):
    w = _wid()
    k16 = _iota()
    sel = k16 >> 3
    lane8 = k16 & 7

    def ch(j, _):
        base = pl.multiple_of(w * SPAN + j * CH, 8)
        pltpu.sync_copy(srcp.at[pl.ds(base, CH)], src_c)
        pltpu.sync_copy(dstp.at[pl.ds(base, CH)], dst_c)
        g1 = pltpu.async_copy(aa.at[src_c], asrc_r, sem)
        g2 = pltpu.async_copy(td.at[dst_c], adst_r, sem)
        g1.wait()
        g2.wait()

        def pair(i, _):
            for u in range(4):
                k = i * 4 + u
                ep = 2 * k + sel
                a_s = plsc.load_gather(asrc_r, [ep, lane8])
                a_d = plsc.load_gather(adst_r, [ep, lane8])
                dv = plsc.load_gather(adst_r, [ep, 8 + lane8])
                z = a_s + a_d
                z = jnp.where(z > 0, z, 0.2 * z)
                ko = pl.multiple_of(k * 16, 16)
                an_c[pl.ds(ko, 16)] = jnp.exp(z) / (dv + 1e-16)
            return 0
        lax.fori_loop(0, CH // 8, pair, 0)
        ofs = pl.multiple_of((w * CPT + j) * CH * 8, CH * 8)
        pltpu.sync_copy(an_c, a1r.at[pl.ds(ofs, CH * 8)])
        return 0
    lax.fori_loop(0, CPT, ch, 0)


_alpha1 = pl.kernel(
    _alpha1_body,
    out_type=jax.ShapeDtypeStruct((NW * CPT * CH * 8,), jnp.float32),
    mesh=_mesh,
    compiler_params=_sc_params,
    scratch_types=[pltpu.VMEM((CH,), jnp.int32),
                   pltpu.VMEM((CH,), jnp.int32),
                   pltpu.VMEM((CH, 16), jnp.float32),
                   pltpu.VMEM((CH, 16), jnp.float32),
                   pltpu.VMEM((CH * 8,), jnp.float32),
                   pltpu.SemaphoreType.DMA],
)


def _msg2_body(ssrc, sdst, seid, nstart, ntot, aa, htab,
               msg, a2out, araw,
               nst_v, ntt_v, aa_loc, denom, src_c, dst_c, eid_c,
               asrc_r, araw_c, an_c, ai, av, hrows, acc, sem):
    w = _wid()
    nbase = pl.multiple_of(w * BSZ, BSZ)
    k16 = _iota()
    z16 = jnp.zeros((16,), jnp.float32)
    zi16 = jnp.zeros((16,), jnp.int32)
    one16 = jnp.ones((16,), jnp.int32)
    pltpu.sync_copy(nstart.at[pl.ds(nbase, BSZ)], nst_v)
    pltpu.sync_copy(ntot.at[pl.ds(nbase, BSZ)], ntt_v)
    pltpu.sync_copy(aa.at[pl.ds(nbase, BSZ)], aa_loc)

    def cnt(i, a):
        return a + ntt_v[pl.ds(i * 16, 16)]
    necnt = jnp.sum(lax.fori_loop(0, BSZ // 16, cnt,
                                  jnp.zeros((16,), jnp.int32)))
    bstart = pl.multiple_of(nst_v[pl.ds(0, 16)][0], CH)
    nchunks = (necnt + CH - 1) // CH

    def zden(i, _):
        denom[pl.ds(i * 16, 16)] = z16
        return 0
    lax.fori_loop(0, BSZ // 16, zden, 0)

    def zacc(i, _):
        for c in range(4):
            acc[i, pl.ds(c * 16, 16)] = z16
        return 0
    lax.fori_loop(0, BSZ, zacc, 0)

    def load_chunk(j, with_eid):
        cofs = pl.multiple_of(bstart + j * CH, CH)
        pltpu.sync_copy(ssrc.at[pl.ds(cofs, CH)], src_c)
        pltpu.sync_copy(sdst.at[pl.ds(cofs, CH)], dst_c.at[pl.ds(0, CH)])
        if with_eid:
            pltpu.sync_copy(seid.at[pl.ds(cofs, CH)], eid_c)
        clen = jnp.minimum(CH, necnt - j * CH)

        def san(v, _):
            m = (v * 16 + k16) < clen
            sl = pl.ds(v * 16, 16)
            src_c[sl] = jnp.where(m, src_c[sl], 0)
            dst_c[sl] = jnp.where(m, dst_c[sl], nbase)
            if with_eid:
                eid_c[sl] = jnp.where(m, eid_c[sl], ET)
            return 0
        lax.fori_loop(0, CH // 16, san, 0)
        return cofs

    def s1(j, _):
        cofs = load_chunk(j, True)
        pltpu.async_copy(aa.at[src_c], asrc_r, sem).wait()

        def vec(v, _):
            sl = pl.ds(v * 16, 16)
            d16 = dst_c[sl]
            dl = d16 - nbase
            eid16 = eid_c[sl]
            a_s = plsc.load_gather(asrc_r, [v * 16 + k16, zi16])
            a_d = plsc.load_gather(aa_loc, [dl, one16])
            z = a_s + a_d
            z = jnp.where(z > 0, z, 0.2 * z)
            al = jnp.where(eid16 < ET, jnp.exp(z), 0.0)
            araw_c[sl] = al
            prev = _take16(d16, jnp.maximum(k16 - 1, 0))
            is_start = (k16 == 0) | (d16 != prev)
            csum = plsc.cumsum(al)
            spos = plsc.cummax(jnp.where(is_start, k16, 0))
            base_excl = jnp.where(spos > 0,
                                  _take16(csum, jnp.maximum(spos - 1, 0)),
                                  0.0)
            nxt = _take16(d16, jnp.minimum(k16 + 1, 15))
            is_end = (k16 == 15) | (d16 != nxt)
            plsc.addupdate_scatter(denom, [dl], csum - base_excl,
                                   mask=is_end)
            return 0
        lax.fori_loop(0, CH // 16, vec, 0)
        pltpu.sync_copy(araw_c, araw.at[pl.ds(cofs, CH)])
        return 0
    lax.fori_loop(0, nchunks, s1, 0)

    def s2(j, _):
        cofs = load_chunk(j, True)
        pltpu.sync_copy(araw.at[pl.ds(cofs, CH)], araw_c)
        pltpu.async_copy(htab.at[src_c], hrows, sem).wait()

        def vec(v, _):
            sl = pl.ds(v * 16, 16)
            dl = dst_c[sl] - nbase
            dv = plsc.load_gather(denom, [dl])
            aln = araw_c[sl] / (dv + 1e-16)
            an_c[sl] = aln
            r = v // 8
            c = (v % 8) * 16
            ai[r, pl.ds(c, 16)] = eid_c[sl]
            av[r, pl.ds(c, 16)] = aln
            return 0
        lax.fori_loop(0, CH // 16, vec, 0)
        d0 = pltpu.async_copy(av.at[0], a2out.at[ai.at[0]], sem)
        d1 = pltpu.async_copy(av.at[1], a2out.at[ai.at[1]], sem)

        def edge(i, _):
            io = pl.multiple_of(i * 8, 8)
            dvec = dst_c[pl.ds(io, 16)]
            avec = an_c[pl.ds(io, 16)]
            for u in range(8):
                e = i * 8 + u
                dl = dvec[u] - nbase
                ab = _take16(avec, jnp.full((16,), u, jnp.int32))
                for c in range(4):
                    sl = pl.ds(c * 16, 16)
                    acc[dl, sl] += ab * hrows[e, sl]
            return 0
        lax.fori_loop(0, CH // 8, edge, 0)
        d0.wait()
        d1.wait()
        return 0
    lax.fori_loop(0, nchunks, s2, 0)
    pltpu.sync_copy(acc, msg.at[pl.ds(nbase, BSZ)])


_msg2 = pl.kernel(
    _msg2_body,
    out_type=(jax.ShapeDtypeStruct((NPAD, 64), jnp.float32),
              jax.ShapeDtypeStruct((EPAD,), jnp.float32),
              jax.ShapeDtypeStruct((ECAP,), jnp.float32)),
    mesh=_mesh,
    compiler_params=_sc_params,
    scratch_types=[pltpu.VMEM((BSZ,), jnp.int32),
                   pltpu.VMEM((BSZ,), jnp.int32),
                   pltpu.VMEM((BSZ, 16), jnp.float32),
                   pltpu.VMEM((BSZ,), jnp.float32),
                   pltpu.VMEM((CH,), jnp.int32),
                   pltpu.VMEM((CH + 16,), jnp.int32),
                   pltpu.VMEM((CH,), jnp.int32),
                   pltpu.VMEM((CH, 16), jnp.float32),
                   pltpu.VMEM((CH,), jnp.float32),
                   pltpu.VMEM((CH + 16,), jnp.float32),
                   pltpu.VMEM((2, 128), jnp.int32),
                   pltpu.VMEM((2, 128), jnp.float32),
                   pltpu.VMEM((CH, 64), jnp.float32),
                   pltpu.VMEM((BSZ, 64), jnp.float32),
                   pltpu.SemaphoreType.DMA],
)


def _mm_kernel(x_ref, w_ref, o_ref):
    o_ref[...] = jnp.dot(x_ref[...], w_ref[...],
                         preferred_element_type=jnp.float32)


def _matmul(x, w, block_m=1000):
    M, K = x.shape
    _, Nc = w.shape
    return pl.pallas_call(
        _mm_kernel,
        grid=(M // block_m,),
        in_specs=[pl.BlockSpec((block_m, K), lambda i: (i, 0)),
                  pl.BlockSpec((K, Nc), lambda i: (0, 0))],
        out_specs=pl.BlockSpec((block_m, Nc), lambda i: (i, 0)),
        out_shape=jax.ShapeDtypeStruct((M, Nc), jnp.float32),
    )(x, w)


def _mm_elu_kernel(m_ref, b_ref, w_ref, o_ref):
    v = m_ref[...] + b_ref[...]
    v = jnp.where(v > 0, v, jnp.exp(v) - 1.0)
    o_ref[...] = jnp.dot(v, w_ref[...], preferred_element_type=jnp.float32)


def _mm_elu(msg, b, w, block_m=1000):
    K = msg.shape[1]
    Nc = w.shape[1]
    return pl.pallas_call(
        _mm_elu_kernel,
        grid=(N // block_m,),
        in_specs=[pl.BlockSpec((block_m, K), lambda i: (i, 0)),
                  pl.BlockSpec((1, K), lambda i: (0, 0)),
                  pl.BlockSpec((K, Nc), lambda i: (0, 0))],
        out_specs=pl.BlockSpec((block_m, Nc), lambda i: (i, 0)),
        out_shape=jax.ShapeDtypeStruct((N, Nc), jnp.float32),
    )(msg, b.reshape(1, K), w)


def _lsm_kernel(m_ref, b_ref, o_ref):
    z = m_ref[...] + b_ref[...]
    mx = jnp.max(z, axis=1, keepdims=True)
    s = jnp.log(jnp.sum(jnp.exp(z - mx), axis=1, keepdims=True))
    o_ref[...] = z - mx - s


def _logsoftmax(msg2, b2, block_m=1000):
    return pl.pallas_call(
        _lsm_kernel,
        grid=(N // block_m,),
        in_specs=[pl.BlockSpec((block_m, 64), lambda i: (i, 0)),
                  pl.BlockSpec((1, 64), lambda i: (0, 0))],
        out_specs=pl.BlockSpec((block_m, 64), lambda i: (i, 0)),
        out_shape=jax.ShapeDtypeStruct((N, 64), jnp.float32),
    )(msg2, b2.reshape(1, 64))


def kernel(x, edge_index, W1, att_src1, att_dst1, b1, W2, att_src2,
           att_dst2, b2):
    loop = jnp.arange(N, dtype=edge_index.dtype)
    src = jnp.concatenate(
        [edge_index[0], loop, jnp.zeros((EPAD - ET,), edge_index.dtype)])
    dst = jnp.concatenate(
        [edge_index[1], loop, jnp.full((EPAD - ET,), N - 1, edge_index.dtype)])

    counts = _hist(dst)
    ssrc_r, sdst_r, seid_r, nstart, ntot = _place(src, dst, counts)

    h1mat = _matmul(x, W1)
    eye8 = jnp.eye(8, dtype=jnp.float32)
    A_s = (att_src1.reshape(8, 128)[:, :, None] *
           eye8[:, None, :]).reshape(1024, 8)
    A_d = (att_dst1.reshape(8, 128)[:, :, None] *
           eye8[:, None, :]).reshape(1024, 8)
    aa1 = _matmul(h1mat, jnp.concatenate([A_s, A_d], axis=1))
    aa1p = jnp.pad(aa1, ((0, NPAD - N), (0, 0)))
    htab = h1mat.reshape(N * 8, 128)
    msg, _araw1, dn1f = _msg1(ssrc_r, sdst_r, seid_r, nstart, ntot,
                              aa1p, htab)
    td1 = jnp.concatenate([aa1p[:, 8:16], dn1f.reshape(NPAD, 8)], axis=1)
    srcp2 = jnp.pad(src, (0, 256))
    dstp2 = jnp.pad(dst, (0, 256))
    a1r = _alpha1(srcp2, dstp2, aa1p, td1)
    alpha1 = a1r.reshape(NW, CPT * CH * 8)[:, :SPAN * 8].reshape(EPAD, 8)[:ET]

    h2pre = _mm_elu(msg, b1, W2)
    A2 = jnp.concatenate([att_src2.reshape(64, 1), att_dst2.reshape(64, 1),
                          jnp.zeros((64, 14), jnp.float32)], axis=1)
    aa2p = jnp.pad(_matmul(h2pre, A2), ((0, NPAD - N), (0, 0)))
    msg2, a2flat, _araw2 = _msg2(ssrc_r, sdst_r, seid_r, nstart, ntot,
                                 aa2p, h2pre)
    logp = _logsoftmax(msg2, b2)
    alpha2 = a2flat[:ET].reshape(ET, 1)
    return (logp, alpha1, alpha2)


# speedup vs baseline: 2.2138x; 1.0561x over previous
import functools

import jax
import jax.numpy as jnp
from jax import lax
from jax.experimental import pallas as pl
from jax.experimental.pallas import tpu as pltpu
from jax.experimental.pallas import tpu_sc as plsc

N = 10000
E = 320000
ET = E + N
NW = 32
SPAN = 10320
EPAD = NW * SPAN
NB = 32
BSZ = 320
NPAD = NB * BSZ
CH = 256
ECAP = EPAD + NB * CH
EALLOC = ECAP + CH
DUMMY = ECAP + 128
NVEC = SPAN // 16
SROWS = (SPAN + 127) // 128

_mesh = plsc.VectorSubcoreMesh(core_axis_name="c", subcore_axis_name="s")
_sc_params = pltpu.CompilerParams(needs_layout_passes=False,
                                  use_tc_tiling_on_sc=False)


def _wid():
    return lax.axis_index("s") * 2 + lax.axis_index("c")


def _iota():
    return lax.iota(jnp.int32, 16)


def _take16(x, idx):
    return lax.gather(
        x, idx[:, None],
        lax.GatherDimensionNumbers(offset_dims=(), collapsed_slice_dims=(0,),
                                   start_index_map=(0,)),
        (1,), mode=lax.GatherScatterMode.PROMISE_IN_BOUNDS)


def _group_info(sd):
    k = _iota()
    prev = _take16(sd, jnp.maximum(k - 1, 0))
    is_start = (k == 0) | (sd != prev)
    startpos = plsc.cummax(jnp.where(is_start, k, 0))
    rank = k - startpos
    nxt = _take16(sd, jnp.minimum(k + 1, 15))
    is_end = (k == 15) | (sd != nxt)
    return rank, is_end


def _hist_body(dst_hbm, counts_hbm, dbuf, cnt):
    w = _wid()
    z16 = jnp.zeros((16,), jnp.int32)

    def zloop(i, _):
        cnt[pl.ds(i * 16, 16)] = z16
        return 0
    lax.fori_loop(0, NPAD // 16, zloop, 0)
    pltpu.sync_copy(dst_hbm.at[pl.ds(w * SPAN, SPAN)], dbuf)

    def body(i, _):
        d = dbuf[pl.ds(i * 16, 16)]
        sd, _sl = plsc.sort_key_val(d, _iota())
        rank, is_end = _group_info(sd)
        plsc.addupdate_scatter(cnt, [sd], rank + 1, mask=is_end)
        return 0
    lax.fori_loop(0, NVEC, body, 0)
    pltpu.sync_copy(cnt, counts_hbm.at[w])


_hist = pl.kernel(
    _hist_body,
    out_type=jax.ShapeDtypeStruct((NW, NPAD), jnp.int32),
    mesh=_mesh,
    compiler_params=_sc_params,
    scratch_types=[pltpu.VMEM((SPAN,), jnp.int32),
                   pltpu.VMEM((NPAD,), jnp.int32)],
)


def _place_body(src_hbm, dst_hbm, counts_hbm,
                ssrc_hbm, sdst_hbm, seid_hbm, nstart_hbm, ntot_hbm,
                sbuf, dbuf, tmp, tot, below, nstart_v, bstart_v,
                posb, soutb, doutb, eoutb, sem):
    w = _wid()
    z16 = jnp.zeros((16,), jnp.int32)
    nv = NPAD // 16

    def zloop(i, _):
        tot[pl.ds(i * 16, 16)] = z16
        below[pl.ds(i * 16, 16)] = z16
        return 0
    lax.fori_loop(0, nv, zloop, 0)

    def agg(t, _):
        pltpu.sync_copy(counts_hbm.at[t], tmp)

        def add(i, _):
            v = tmp[pl.ds(i * 16, 16)]
            tot[pl.ds(i * 16, 16)] += v
            return 0
        lax.fori_loop(0, nv, add, 0)

        @pl.when(t < w)
        def _():
            def addb(i, _):
                below[pl.ds(i * 16, 16)] += tmp[pl.ds(i * 16, 16)]
                return 0
            lax.fori_loop(0, nv, addb, 0)
        return 0
    lax.fori_loop(0, NW, agg, 0)

    def btot(b, run):
        def acc(i, a):
            return a + tot[pl.ds(b * BSZ + i * 16, 16)]
        a16 = lax.fori_loop(0, BSZ // 16, acc, z16)
        bt = jnp.sum(a16)
        bstart_v[b] = run
        return run + ((bt + CH - 1) & -CH)
    lax.fori_loop(0, NB, btot, jnp.int32(0))

    def nloop(b, _):
        bs = bstart_v[b]

        def inner(i, run):
            v = tot[pl.ds(b * BSZ + i * 16, 16)]
            c = plsc.cumsum(v)
            nstart_v[pl.ds(b * BSZ + i * 16, 16)] = c - v + run
            return run + jnp.sum(v)
        lax.fori_loop(0, BSZ // 16, inner, bs)
        return 0
    lax.fori_loop(0, NB, nloop, 0)

    def curs(i, _):
        below[pl.ds(i * 16, 16)] += nstart_v[pl.ds(i * 16, 16)]
        return 0
    lax.fori_loop(0, nv, curs, 0)

    @pl.when(w == 0)
    def _():
        pltpu.sync_copy(nstart_v, nstart_hbm)
        pltpu.sync_copy(tot, ntot_hbm)

    pltpu.sync_copy(src_hbm.at[pl.ds(w * SPAN, SPAN)], sbuf)
    pltpu.sync_copy(dst_hbm.at[pl.ds(w * SPAN, SPAN)], dbuf)
    dm16 = jnp.full((16,), DUMMY, jnp.int32)
    for c in range(8):
        posb[SROWS - 1, pl.ds(c * 16, 16)] = dm16

    def place(i, _):
        d = dbuf[pl.ds(i * 16, 16)]
        s = sbuf[pl.ds(i * 16, 16)]
        sd, sl = plsc.sort_key_val(d, _iota())
        sp = _take16(s, sl)
        ep = w * SPAN + i * 16 + sl
        rank, is_end = _group_info(sd)
        pos = plsc.load_gather(below, [sd]) + rank
        plsc.store_scatter(below, [sd], pos + 1, mask=is_end)
        r = i // 8
        cofs = (i % 8) * 16
        posb[r, pl.ds(cofs, 16)] = pos
        soutb[r, pl.ds(cofs, 16)] = sp
        doutb[r, pl.ds(cofs, 16)] = sd
        eoutb[r, pl.ds(cofs, 16)] = ep
        return 0
    lax.fori_loop(0, NVEC, place, 0)

    def scat(j, _):
        a = pltpu.async_copy(soutb.at[j], ssrc_hbm.at[posb.at[j]], sem)
        b = pltpu.async_copy(doutb.at[j], sdst_hbm.at[posb.at[j]], sem)
        c = pltpu.async_copy(eoutb.at[j], seid_hbm.at[posb.at[j]], sem)
        a.wait()
        b.wait()
        c.wait()
        return 0
    lax.fori_loop(0, SROWS, scat, 0)


_place = pl.kernel(
    _place_body,
    out_type=(jax.ShapeDtypeStruct((EALLOC,), jnp.int32),
              jax.ShapeDtypeStruct((EALLOC,), jnp.int32),
              jax.ShapeDtypeStruct((EALLOC,), jnp.int32),
              jax.ShapeDtypeStruct((NPAD,), jnp.int32),
              jax.ShapeDtypeStruct((NPAD,), jnp.int32)),
    mesh=_mesh,
    compiler_params=_sc_params,
    scratch_types=[pltpu.VMEM((SPAN,), jnp.int32),
                   pltpu.VMEM((SPAN,), jnp.int32),
                   pltpu.VMEM((NPAD,), jnp.int32),
                   pltpu.VMEM((NPAD,), jnp.int32),
                   pltpu.VMEM((NPAD,), jnp.int32),
                   pltpu.VMEM((NPAD,), jnp.int32),
                   pltpu.SMEM((NB,), jnp.int32),
                   pltpu.VMEM((SROWS, 128), jnp.int32),
                   pltpu.VMEM((SROWS, 128), jnp.int32),
                   pltpu.VMEM((SROWS, 128), jnp.int32),
                   pltpu.VMEM((SROWS, 128), jnp.int32),
                   pltpu.SemaphoreType.DMA],
)


def _msg1_body(ssrc, sdst, seid, nstart, ntot, aa, htab,
               msg, araw, dnout,
               nst_v, ntt_v, aa_loc, denom, src_c, dst_c, eid_c,
               asrc_r, araw_c, gidx, hrows, acc, sem):
    w = _wid()
    nbase = pl.multiple_of(w * BSZ, BSZ)
    k16 = _iota()
    sel = k16 >> 3
    lane8 = k16 & 7
    z16 = jnp.zeros((16,), jnp.float32)
    pltpu.sync_copy(nstart.at[pl.ds(nbase, BSZ)], nst_v)
    pltpu.sync_copy(ntot.at[pl.ds(nbase, BSZ)], ntt_v)
    pltpu.sync_copy(aa.at[pl.ds(nbase, BSZ)], aa_loc)

    def cnt(i, a):
        return a + ntt_v[pl.ds(i * 16, 16)]
    necnt = jnp.sum(lax.fori_loop(0, BSZ // 16, cnt,
                                  jnp.zeros((16,), jnp.int32)))
    bstart = pl.multiple_of(nst_v[pl.ds(0, 16)][0], CH)
    nchunks = (necnt + CH - 1) // CH

    def zden(i, _):
        denom[pl.ds(i * 16, 16)] = z16
        return 0
    lax.fori_loop(0, (BSZ * 8) // 16, zden, 0)

    def load_chunk(j, with_eid):
        cofs = pl.multiple_of(bstart + j * CH, CH)
        pltpu.sync_copy(ssrc.at[pl.ds(cofs, CH)], src_c)
        pltpu.sync_copy(sdst.at[pl.ds(cofs, CH)], dst_c.at[pl.ds(0, CH)])
        if with_eid:
            pltpu.sync_copy(seid.at[pl.ds(cofs, CH)], eid_c)
        clen = jnp.minimum(CH, necnt - j * CH)

        def san(v, _):
            m = (v * 16 + k16) < clen
            sl = pl.ds(v * 16, 16)
            src_c[sl] = jnp.where(m, src_c[sl], 0)
            dst_c[sl] = jnp.where(m, dst_c[sl], nbase)
            if with_eid:
                eid_c[sl] = jnp.where(m, eid_c[sl], ET)
            return 0
        lax.fori_loop(0, CH // 16, san, 0)
        return cofs

    def s1(j, _):
        cofs = load_chunk(j, True)
        pltpu.async_copy(aa.at[src_c], asrc_r, sem).wait()

        def pair(i, _):
            for u in range(4):
                k = i * 4 + u
                ep = 2 * k + sel
                dl = plsc.load_gather(dst_c, [ep]) - nbase
                eidp = plsc.load_gather(eid_c, [ep])
                a_s = plsc.load_gather(asrc_r, [ep, lane8])
                a_d = plsc.load_gather(aa_loc, [dl, 8 + lane8])
                z = a_s + a_d
                z = jnp.where(z > 0, z, 0.2 * z)
                al = jnp.where(eidp < ET, jnp.exp(z), 0.0)
                araw_c[pl.ds(pl.multiple_of(k * 16, 16), 16)] = al
                addr = dl * 8 + lane8
                plsc.addupdate_scatter(denom, [addr], al, mask=(sel == 0))
                plsc.addupdate_scatter(denom, [addr], al, mask=(sel == 1))
            return 0
        lax.fori_loop(0, CH // 8, pair, 0)
        pltpu.sync_copy(araw_c.at[pl.ds(0, CH * 8)],
                        araw.at[pl.ds(cofs * 8, CH * 8)])
        return 0
    lax.fori_loop(0, nchunks, s1, 0)
    pltpu.sync_copy(denom, dnout.at[pl.ds(nbase * 8, BSZ * 8)])


    def sh(h, _):
        hfull = jnp.full((16,), h, jnp.int32)

        def zacc(i, _):
            for c in range(8):
                acc[i, pl.ds(c * 16, 16)] = z16
            return 0
        lax.fori_loop(0, BSZ, zacc, 0)

        def s3(j, _):
            cofs = load_chunk(j, False)

            def gi(v, _):
                sl = pl.ds(v * 16, 16)
                gidx[sl] = src_c[sl] * 8 + h
                return 0
            lax.fori_loop(0, CH // 16, gi, 0)
            cp = pltpu.async_copy(htab.at[gidx], hrows, sem)
            pltpu.sync_copy(araw.at[pl.ds(cofs * 8, CH * 8)],
                            araw_c.at[pl.ds(0, CH * 8)])
            cp.wait()

            def edge(i, _):
                dvec = dst_c[pl.ds(i * 8, 16)]
                for u in range(8):
                    e = i * 8 + u
                    dl = dvec[u] - nbase
                    ao = pl.multiple_of(e * 8, 8)
                    ab = _take16(araw_c[pl.ds(ao, 16)], hfull)
                    for c in range(8):
                        sl = pl.ds(c * 16, 16)
                        plsc.addupdate(acc.at[dl, sl], ab * hrows[e, sl])
                return 0
            lax.fori_loop(0, CH // 8, edge, 0)
            return 0
        lax.fori_loop(0, nchunks, s3, 0)

        def nsc(n, _):
            dv = plsc.load_gather(denom,
                                  [jnp.full((16,), n, jnp.int32) * 8 + hfull])
            inv = 1.0 / (dv + 1e-16)
            for c in range(8):
                sl = pl.ds(c * 16, 16)
                acc[n, sl] *= inv
            return 0
        lax.fori_loop(0, BSZ, nsc, 0)
        pltpu.sync_copy(acc, msg.at[pl.ds(nbase, BSZ), pl.ds(h * 128, 128)])
        return 0
    lax.fori_loop(0, 8, sh, 0)


_msg1 = pl.kernel(
    _msg1_body,
    out_type=(jax.ShapeDtypeStruct((NPAD, 1024), jnp.float32),
              jax.ShapeDtypeStruct((ECAP * 8,), jnp.float32),
              jax.ShapeDtypeStruct((NPAD * 8,), jnp.float32)),
    mesh=_mesh,
    compiler_params=_sc_params,
    scratch_types=[pltpu.VMEM((BSZ,), jnp.int32),
                   pltpu.VMEM((BSZ,), jnp.int32),
                   pltpu.VMEM((BSZ, 16), jnp.float32),
                   pltpu.VMEM((BSZ * 8,), jnp.float32),
                   pltpu.VMEM((CH,), jnp.int32),
                   pltpu.VMEM((CH + 16,), jnp.int32),
                   pltpu.VMEM((CH,), jnp.int32),
                   pltpu.VMEM((CH, 16), jnp.float32),
                   pltpu.VMEM((CH * 8 + 16,), jnp.float32),
                   pltpu.VMEM((CH,), jnp.int32),
                   pltpu.VMEM((CH, 128), jnp.float32),
                   pltpu.VMEM((BSZ, 128), jnp.float32),
                   pltpu.SemaphoreType.DMA],
)


CPT = (SPAN + CH - 1) // CH


def _alpha1_body(srcp, dstp, aa, td, a1r, src_c, dst_c, asrc_r, adst_r,
                 an_c, sem):
    w = _wid()
    k16 = _iota()
    sel = k16 >> 3
    lane8 = k16 & 7

    def ch(j, _):
        base = pl.multiple_of(w * SPAN + j * CH, 8)
        pltpu.sync_copy(srcp.at[pl.ds(base, CH)], src_c)
        pltpu.sync_copy(dstp.at[pl.ds(base, CH)], dst_c)
        g1 = pltpu.async_copy(aa.at[src_c], asrc_r, sem)
        g2 = pltpu.async_copy(td.at[dst_c], adst_r, sem)
        g1.wait()
        g2.wait()

        def pair(i, _):
            for u in range(4):
                k = i * 4 + u
                ep = 2 * k + sel
                a_s = plsc.load_gather(asrc_r, [ep, lane8])
                a_d = plsc.load_gather(adst_r, [ep, lane8])
                dv = plsc.load_gather(adst_r, [ep, 8 + lane8])
                z = a_s + a_d
                z = jnp.where(z > 0, z, 0.2 * z)
                ko = pl.multiple_of(k * 16, 16)
                an_c[pl.ds(ko, 16)] = jnp.exp(z) / (dv + 1e-16)
            return 0
        lax.fori_loop(0, CH // 8, pair, 0)
        ofs = pl.multiple_of((w * CPT + j) * CH * 8, CH * 8)
        pltpu.sync_copy(an_c, a1r.at[pl.ds(ofs, CH * 8)])
        return 0
    lax.fori_loop(0, CPT, ch, 0)


_alpha1 = pl.kernel(
    _alpha1_body,
    out_type=jax.ShapeDtypeStruct((NW * CPT * CH * 8,), jnp.float32),
    mesh=_mesh,
    compiler_params=_sc_params,
    scratch_types=[pltpu.VMEM((CH,), jnp.int32),
                   pltpu.VMEM((CH,), jnp.int32),
                   pltpu.VMEM((CH, 16), jnp.float32),
                   pltpu.VMEM((CH, 16), jnp.float32),
                   pltpu.VMEM((CH * 8,), jnp.float32),
                   pltpu.SemaphoreType.DMA],
)


def _alpha2_body(srcp, dstp, aa, td, a2r, src_c, dst_c, asrc_r, adst_r,
                 an_c, sem):
    w = _wid()
    k16 = _iota()
    zi16 = jnp.zeros((16,), jnp.int32)
    one16 = jnp.ones((16,), jnp.int32)

    def ch(j, _):
        base = pl.multiple_of(w * SPAN + j * CH, 8)
        pltpu.sync_copy(srcp.at[pl.ds(base, CH)], src_c)
        pltpu.sync_copy(dstp.at[pl.ds(base, CH)], dst_c)
        g1 = pltpu.async_copy(aa.at[src_c], asrc_r, sem)
        g2 = pltpu.async_copy(td.at[dst_c], adst_r, sem)
        g1.wait()
        g2.wait()

        def vec(v, _):
            sl = pl.ds(v * 16, 16)
            a_s = plsc.load_gather(asrc_r, [v * 16 + k16, zi16])
            a_d = plsc.load_gather(adst_r, [v * 16 + k16, zi16])
            dv = plsc.load_gather(adst_r, [v * 16 + k16, one16])
            z = a_s + a_d
            z = jnp.where(z > 0, z, 0.2 * z)
            an_c[sl] = jnp.exp(z) / (dv + 1e-16)
            return 0
        lax.fori_loop(0, CH // 16, vec, 0)
        ofs = pl.multiple_of((w * CPT + j) * CH, CH)
        pltpu.sync_copy(an_c, a2r.at[pl.ds(ofs, CH)])
        return 0
    lax.fori_loop(0, CPT, ch, 0)


_alpha2 = pl.kernel(
    _alpha2_body,
    out_type=jax.ShapeDtypeStruct((NW * CPT * CH,), jnp.float32),
    mesh=_mesh,
    compiler_params=_sc_params,
    scratch_types=[pltpu.VMEM((CH,), jnp.int32),
                   pltpu.VMEM((CH,), jnp.int32),
                   pltpu.VMEM((CH, 16), jnp.float32),
                   pltpu.VMEM((CH, 16), jnp.float32),
                   pltpu.VMEM((CH,), jnp.float32),
                   pltpu.SemaphoreType.DMA],
)


def _msg2_body(ssrc, sdst, seid, nstart, ntot, aa, htab,
               msg, araw, dnout,
               nst_v, ntt_v, aa_loc, denom, src_c, dst_c, eid_c,
               asrc_r, araw_c, hrows, acc, sem):
    w = _wid()
    nbase = pl.multiple_of(w * BSZ, BSZ)
    k16 = _iota()
    z16 = jnp.zeros((16,), jnp.float32)
    zi16 = jnp.zeros((16,), jnp.int32)
    one16 = jnp.ones((16,), jnp.int32)
    pltpu.sync_copy(nstart.at[pl.ds(nbase, BSZ)], nst_v)
    pltpu.sync_copy(ntot.at[pl.ds(nbase, BSZ)], ntt_v)
    pltpu.sync_copy(aa.at[pl.ds(nbase, BSZ)], aa_loc)

    def cnt(i, a):
        return a + ntt_v[pl.ds(i * 16, 16)]
    necnt = jnp.sum(lax.fori_loop(0, BSZ // 16, cnt,
                                  jnp.zeros((16,), jnp.int32)))
    bstart = pl.multiple_of(nst_v[pl.ds(0, 16)][0], CH)
    nchunks = (necnt + CH - 1) // CH

    def zden(i, _):
        denom[pl.ds(i * 16, 16)] = z16
        return 0
    lax.fori_loop(0, BSZ // 16, zden, 0)

    def zacc(i, _):
        for c in range(4):
            acc[i, pl.ds(c * 16, 16)] = z16
        return 0
    lax.fori_loop(0, BSZ, zacc, 0)

    def load_chunk(j, with_eid):
        cofs = pl.multiple_of(bstart + j * CH, CH)
        pltpu.sync_copy(ssrc.at[pl.ds(cofs, CH)], src_c)
        pltpu.sync_copy(sdst.at[pl.ds(cofs, CH)], dst_c.at[pl.ds(0, CH)])
        if with_eid:
            pltpu.sync_copy(seid.at[pl.ds(cofs, CH)], eid_c)
        clen = jnp.minimum(CH, necnt - j * CH)

        def san(v, _):
            m = (v * 16 + k16) < clen
            sl = pl.ds(v * 16, 16)
            src_c[sl] = jnp.where(m, src_c[sl], 0)
            dst_c[sl] = jnp.where(m, dst_c[sl], nbase)
            if with_eid:
                eid_c[sl] = jnp.where(m, eid_c[sl], ET)
            return 0
        lax.fori_loop(0, CH // 16, san, 0)
        return cofs

    def s1(j, _):
        cofs = load_chunk(j, True)
        pltpu.async_copy(aa.at[src_c], asrc_r, sem).wait()

        def vec(v, _):
            sl = pl.ds(v * 16, 16)
            d16 = dst_c[sl]
            dl = d16 - nbase
            eid16 = eid_c[sl]
            a_s = plsc.load_gather(asrc_r, [v * 16 + k16, zi16])
            a_d = plsc.load_gather(aa_loc, [dl, one16])
            z = a_s + a_d
            z = jnp.where(z > 0, z, 0.2 * z)
            al = jnp.where(eid16 < ET, jnp.exp(z), 0.0)
            araw_c[sl] = al
            prev = _take16(d16, jnp.maximum(k16 - 1, 0))
            is_start = (k16 == 0) | (d16 != prev)
            csum = plsc.cumsum(al)
            spos = plsc.cummax(jnp.where(is_start, k16, 0))
            base_excl = jnp.where(spos > 0,
                                  _take16(csum, jnp.maximum(spos - 1, 0)),
                                  0.0)
            nxt = _take16(d16, jnp.minimum(k16 + 1, 15))
            is_end = (k16 == 15) | (d16 != nxt)
            plsc.addupdate_scatter(denom, [dl], csum - base_excl,
                                   mask=is_end)
            return 0
        lax.fori_loop(0, CH // 16, vec, 0)
        pltpu.sync_copy(araw_c.at[pl.ds(0, CH)], araw.at[pl.ds(cofs, CH)])
        return 0
    lax.fori_loop(0, nchunks, s1, 0)
    pltpu.sync_copy(denom, dnout.at[pl.ds(nbase, BSZ)])

    def s2(j, _):
        cofs = load_chunk(j, False)
        pltpu.sync_copy(araw.at[pl.ds(cofs, CH)], araw_c.at[pl.ds(0, CH)])
        pltpu.async_copy(htab.at[src_c], hrows, sem).wait()

        def edge(i, _):
            io = pl.multiple_of(i * 8, 8)
            dvec = dst_c[pl.ds(io, 16)]
            avec = araw_c[pl.ds(io, 16)]
            for u in range(8):
                e = i * 8 + u
                dl = dvec[u] - nbase
                ab = _take16(avec, jnp.full((16,), u, jnp.int32))
                for c in range(4):
                    sl = pl.ds(c * 16, 16)
                    plsc.addupdate(acc.at[dl, sl], ab * hrows[e, sl])
            return 0
        lax.fori_loop(0, CH // 8, edge, 0)
        return 0
    lax.fori_loop(0, nchunks, s2, 0)

    def nsc(n, _):
        dv = plsc.load_gather(denom, [jnp.full((16,), n, jnp.int32)])
        inv = 1.0 / (dv + 1e-16)
        for c in range(4):
            sl = pl.ds(c * 16, 16)
            acc[n, sl] *= inv
        return 0
    lax.fori_loop(0, BSZ, nsc, 0)
    pltpu.sync_copy(acc, msg.at[pl.ds(nbase, BSZ)])


_msg2 = pl.kernel(
    _msg2_body,
    out_type=(jax.ShapeDtypeStruct((NPAD, 64), jnp.float32),
              jax.ShapeDtypeStruct((ECAP,), jnp.float32),
              jax.ShapeDtypeStruct((NPAD,), jnp.float32)),
    mesh=_mesh,
    compiler_params=_sc_params,
    scratch_types=[pltpu.VMEM((BSZ,), jnp.int32),
                   pltpu.VMEM((BSZ,), jnp.int32),
                   pltpu.VMEM((BSZ, 16), jnp.float32),
                   pltpu.VMEM((BSZ,), jnp.float32),
                   pltpu.VMEM((CH,), jnp.int32),
                   pltpu.VMEM((CH + 16,), jnp.int32),
                   pltpu.VMEM((CH,), jnp.int32),
                   pltpu.VMEM((CH, 16), jnp.float32),
                   pltpu.VMEM((CH + 16,), jnp.float32),
                   pltpu.VMEM((CH, 64), jnp.float32),
                   pltpu.VMEM((BSZ, 64), jnp.float32),
                   pltpu.SemaphoreType.DMA],
)


def _mm_kernel(x_ref, w_ref, o_ref):
    o_ref[...] = jnp.dot(x_ref[...], w_ref[...],
                         preferred_element_type=jnp.float32)


def _matmul(x, w, block_m=1000):
    M, K = x.shape
    _, Nc = w.shape
    return pl.pallas_call(
        _mm_kernel,
        grid=(M // block_m,),
        in_specs=[pl.BlockSpec((block_m, K), lambda i: (i, 0)),
                  pl.BlockSpec((K, Nc), lambda i: (0, 0))],
        out_specs=pl.BlockSpec((block_m, Nc), lambda i: (i, 0)),
        out_shape=jax.ShapeDtypeStruct((M, Nc), jnp.float32),
    )(x, w)


def _mm_elu_kernel(m_ref, b_ref, w_ref, o_ref):
    v = m_ref[...] + b_ref[...]
    v = jnp.where(v > 0, v, jnp.exp(v) - 1.0)
    o_ref[...] = jnp.dot(v, w_ref[...], preferred_element_type=jnp.float32)


def _mm_elu(msg, b, w, block_m=1000):
    K = msg.shape[1]
    Nc = w.shape[1]
    return pl.pallas_call(
        _mm_elu_kernel,
        grid=(N // block_m,),
        in_specs=[pl.BlockSpec((block_m, K), lambda i: (i, 0)),
                  pl.BlockSpec((1, K), lambda i: (0, 0)),
                  pl.BlockSpec((K, Nc), lambda i: (0, 0))],
        out_specs=pl.BlockSpec((block_m, Nc), lambda i: (i, 0)),
        out_shape=jax.ShapeDtypeStruct((N, Nc), jnp.float32),
    )(msg, b.reshape(1, K), w)


def _lsm_kernel(m_ref, b_ref, o_ref):
    z = m_ref[...] + b_ref[...]
    mx = jnp.max(z, axis=1, keepdims=True)
    s = jnp.log(jnp.sum(jnp.exp(z - mx), axis=1, keepdims=True))
    o_ref[...] = z - mx - s


def _logsoftmax(msg2, b2, block_m=1000):
    return pl.pallas_call(
        _lsm_kernel,
        grid=(N // block_m,),
        in_specs=[pl.BlockSpec((block_m, 64), lambda i: (i, 0)),
                  pl.BlockSpec((1, 64), lambda i: (0, 0))],
        out_specs=pl.BlockSpec((block_m, 64), lambda i: (i, 0)),
        out_shape=jax.ShapeDtypeStruct((N, 64), jnp.float32),
    )(msg2, b2.reshape(1, 64))


def kernel(x, edge_index, W1, att_src1, att_dst1, b1, W2, att_src2,
           att_dst2, b2):
    loop = jnp.arange(N, dtype=edge_index.dtype)
    src = jnp.concatenate(
        [edge_index[0], loop, jnp.zeros((EPAD - ET,), edge_index.dtype)])
    dst = jnp.concatenate(
        [edge_index[1], loop, jnp.full((EPAD - ET,), N - 1, edge_index.dtype)])

    counts = _hist(dst)
    ssrc_r, sdst_r, seid_r, nstart, ntot = _place(src, dst, counts)

    h1mat = _matmul(x, W1)
    eye8 = jnp.eye(8, dtype=jnp.float32)
    A_s = (att_src1.reshape(8, 128)[:, :, None] *
           eye8[:, None, :]).reshape(1024, 8)
    A_d = (att_dst1.reshape(8, 128)[:, :, None] *
           eye8[:, None, :]).reshape(1024, 8)
    aa1 = _matmul(h1mat, jnp.concatenate([A_s, A_d], axis=1))
    aa1p = jnp.pad(aa1, ((0, NPAD - N), (0, 0)))
    htab = h1mat.reshape(N * 8, 128)
    msg, _araw1, dn1f = _msg1(ssrc_r, sdst_r, seid_r, nstart, ntot,
                              aa1p, htab)
    td1 = jnp.concatenate([aa1p[:, 8:16], dn1f.reshape(NPAD, 8)], axis=1)
    srcp2 = jnp.pad(src, (0, 256))
    dstp2 = jnp.pad(dst, (0, 256))
    a1r = _alpha1(srcp2, dstp2, aa1p, td1)
    alpha1 = a1r.reshape(NW, CPT * CH * 8)[:, :SPAN * 8].reshape(EPAD, 8)[:ET]

    h2pre = _mm_elu(msg, b1, W2)
    A2 = jnp.concatenate([att_src2.reshape(64, 1), att_dst2.reshape(64, 1),
                          jnp.zeros((64, 14), jnp.float32)], axis=1)
    aa2p = jnp.pad(_matmul(h2pre, A2), ((0, NPAD - N), (0, 0)))
    msg2, _araw2, dn2f = _msg2(ssrc_r, sdst_r, seid_r, nstart, ntot,
                               aa2p, h2pre)
    td2 = jnp.concatenate([aa2p[:, 1:2], dn2f[:, None],
                           jnp.zeros((NPAD, 14), jnp.float32)], axis=1)
    a2r = _alpha2(srcp2, dstp2, aa2p, td2)
    logp = _logsoftmax(msg2, b2)
    alpha2 = a2r.reshape(NW, CPT * CH)[:, :SPAN].reshape(EPAD)[:ET]
    alpha2 = alpha2.reshape(ET, 1)
    return (logp, alpha1, alpha2)
